# baseline (speedup 1.0000x reference)
import functools

import jax
import jax.numpy as jnp
from jax import lax
from jax.experimental import pallas as pl
from jax.experimental.pallas import tpu as pltpu
from jax.experimental.pallas import tpu_sc as plsc

DIM = 768
NH = 12
HD = 64
NE = 8
DH = 3072
SEQ = 2048
TILE = 128
NPAD = 3072
NTILES = NPAD // TILE
HSPLIT = 4
DHC = DH // HSPLIT

_DN = (((1,), (0,)), ((), ()))
_DNT = (((1,), (1,)), ((), ()))


def _dot(a, b, dn=_DN):
    return lax.dot_general(a, b, dn, preferred_element_type=jnp.float32)


def _fiota(shape, dim):
    return lax.broadcasted_iota(jnp.int32, shape, dim).astype(jnp.float32)


def _qkv_body(x_ref, g_ref, b_ref, w_ref, wb_ref, q_ref, k_ref, v_ref):
    x = x_ref[...]
    mu = jnp.mean(x, axis=1, keepdims=True)
    var = jnp.mean((x - mu) * (x - mu), axis=1, keepdims=True)
    xn = (x - mu) * lax.rsqrt(var + 1e-5) * g_ref[...] + b_ref[...]
    w = w_ref[...]
    wb = wb_ref[...]
    q_ref[...] = (_dot(xn, w[0:DIM], _DNT) + wb[0, 0:DIM]) * 0.125
    k_ref[...] = _dot(x, w[DIM:2 * DIM], _DNT) + wb[0, DIM:2 * DIM]
    v_ref[...] = _dot(x, w[2 * DIM:3 * DIM], _DNT) + wb[0, 2 * DIM:3 * DIM]


def _qkv_call(xf, g1, b1, w_in, b_in):
    blk = 256
    grid = (SEQ // blk,)
    spec_row = pl.BlockSpec((blk, DIM), lambda i: (i, 0))
    spec_full = pl.BlockSpec((DIM, DIM), lambda i: (0, 0))
    return pl.pallas_call(
        _qkv_body,
        grid=grid,
        in_specs=[
            spec_row,
            pl.BlockSpec((1, DIM), lambda i: (0, 0)),
            pl.BlockSpec((1, DIM), lambda i: (0, 0)),
            pl.BlockSpec((3 * DIM, DIM), lambda i: (0, 0)),
            pl.BlockSpec((1, 3 * DIM), lambda i: (0, 0)),
        ],
        out_specs=[spec_row, spec_row, spec_row],
        out_shape=[jax.ShapeDtypeStruct((SEQ, DIM), jnp.float32)] * 3,
    )(xf, g1, b1, w_in, b_in)


def _attn_body(q_ref, k_ref, v_ref, o_ref):
    q = q_ref[0]
    k = k_ref[0]
    v = v_ref[0]
    s = _dot(q, k, _DNT)
    m = jnp.max(s, axis=1, keepdims=True)
    p = jnp.exp(s - m)
    denom = jnp.sum(p, axis=1, keepdims=True)
    o_ref[0] = _dot(p, v) / denom


def _attn_call(qh, kh, vh):
    qblk = 512
    grid = (NH, SEQ // qblk)
    return pl.pallas_call(
        _attn_body,
        grid=grid,
        in_specs=[
            pl.BlockSpec((1, qblk, HD), lambda h, qb: (h, qb, 0)),
            pl.BlockSpec((1, SEQ, HD), lambda h, qb: (h, 0, 0)),
            pl.BlockSpec((1, SEQ, HD), lambda h, qb: (h, 0, 0)),
        ],
        out_specs=pl.BlockSpec((1, qblk, HD), lambda h, qb: (h, qb, 0)),
        out_shape=jax.ShapeDtypeStruct((NH, SEQ, HD), jnp.float32),
    )(qh, kh, vh)


def _post_body(x_ref, o_ref, wo_ref, bo_ref, g_ref, b_ref, wg_ref, bg_ref,
               xa_ref, x2_ref, lg_ref):
    xa = x_ref[...] + _dot(o_ref[...], wo_ref[...], _DNT) + bo_ref[...]
    xa_ref[...] = xa
    mu = jnp.mean(xa, axis=1, keepdims=True)
    var = jnp.mean((xa - mu) * (xa - mu), axis=1, keepdims=True)
    x2 = (xa - mu) * lax.rsqrt(var + 1e-5) * g_ref[...] + b_ref[...]
    x2_ref[...] = x2
    lg_ref[...] = _dot(x2, wg_ref[...]) + bg_ref[...]


def _post_call(xf, o, wo, bo, g2, b2, wgp, bgp):
    blk = 256
    grid = (SEQ // blk,)
    spec_row = pl.BlockSpec((blk, DIM), lambda i: (i, 0))
    return pl.pallas_call(
        _post_body,
        grid=grid,
        in_specs=[
            spec_row,
            spec_row,
            pl.BlockSpec((DIM, DIM), lambda i: (0, 0)),
            pl.BlockSpec((1, DIM), lambda i: (0, 0)),
            pl.BlockSpec((1, DIM), lambda i: (0, 0)),
            pl.BlockSpec((1, DIM), lambda i: (0, 0)),
            pl.BlockSpec((DIM, 128), lambda i: (0, 0)),
            pl.BlockSpec((1, 128), lambda i: (0, 0)),
        ],
        out_specs=[spec_row, spec_row, pl.BlockSpec((blk, 128), lambda i: (i, 0))],
        out_shape=[
            jax.ShapeDtypeStruct((SEQ, DIM), jnp.float32),
            jax.ShapeDtypeStruct((SEQ, DIM), jnp.float32),
            jax.ShapeDtypeStruct((SEQ, 128), jnp.float32),
        ],
    )(xf, o, wo, bo, g2, b2, wgp, bgp)


def _route_body(lg_ref, dest_ref, gate_ref, gidx_ref, te_ref):
    f32 = jnp.float32
    lg = lg_ref[...]
    m = jnp.max(lg, axis=1, keepdims=True)
    ex = jnp.exp(lg - m)
    sm = jnp.sum(ex, axis=1, keepdims=True)
    gate_ref[...] = 1.0 / sm
    col = _fiota((SEQ, 128), 1)
    idxf = jnp.min(jnp.where(lg == m, col, 1e9), axis=1, keepdims=True)
    onehot = (col == idxf).astype(f32)

    nblk = SEQ // TILE
    rowb = jnp.floor(_fiota((nblk, SEQ), 1) * (1.0 / TILE))
    sel = (rowb == _fiota((nblk, SEQ), 0)).astype(f32)
    bc = _dot(sel, onehot)
    tri_b = (_fiota((nblk, nblk), 0)
             > _fiota((nblk, nblk), 1)).astype(f32)
    cum_bc = _dot(tri_b, bc)
    counts = jnp.sum(bc, axis=0, keepdims=True)
    padded = jnp.floor((counts + (TILE - 1.0)) * (1.0 / TILE)) * TILE
    lt = (_fiota((128, 128), 0)
          < _fiota((128, 128), 1)).astype(f32)
    offs = _dot(padded, lt)
    ends = offs + padded

    tri_t = (_fiota((TILE, TILE), 0)
             > _fiota((TILE, TILE), 1)).astype(f32)
    eye = (_fiota((TILE, TILE), 0)
           == _fiota((TILE, TILE), 1)).astype(f32)
    dest_chunks = []
    dest_rows = []
    for c in range(nblk):
        oh_c = onehot[c * TILE:(c + 1) * TILE]
        pos_c = _dot(tri_t, oh_c) + cum_bc[c:c + 1, :]
        d_c = jnp.sum(oh_c * (offs + pos_c), axis=1, keepdims=True)
        dest_chunks.append(d_c)
        dest_rows.append(jnp.sum(eye * d_c, axis=0, keepdims=True))
    dest = jnp.concatenate(dest_chunks, axis=0)
    dest_row = jnp.concatenate(dest_rows, axis=1)
    dest_ref[...] = dest.astype(jnp.int32)

    tok_row = _fiota((1, SEQ), 1)
    gidx_chunks = []
    for c in range(NTILES):
        slot = c * TILE + _fiota((TILE, 1), 0)
        eq = (dest_row == slot).astype(f32)
        gidx_chunks.append(jnp.sum(eq * tok_row, axis=1, keepdims=True))
    gidx_ref[...] = jnp.concatenate(gidx_chunks, axis=0).astype(jnp.int32)

    lane = _fiota((1, 128), 1)
    trow = _fiota((128, 1), 0) * float(TILE)
    cmp = jnp.where((ends <= trow) & (lane < float(NE)), 1.0, 0.0)
    te = jnp.minimum(jnp.sum(cmp, axis=1, keepdims=True), float(NE - 1))
    te_ref[...] = te.astype(jnp.int32)


def _route_call(logits):
    return pl.pallas_call(
        _route_body,
        in_specs=[pl.BlockSpec((SEQ, 128), lambda: (0, 0))],
        out_specs=[
            pl.BlockSpec((SEQ, 1), lambda: (0, 0)),
            pl.BlockSpec((SEQ, 1), lambda: (0, 0)),
            pl.BlockSpec((NPAD, 1), lambda: (0, 0)),
            pl.BlockSpec((128, 1), lambda: (0, 0)),
        ],
        out_shape=[
            jax.ShapeDtypeStruct((SEQ, 1), jnp.int32),
            jax.ShapeDtypeStruct((SEQ, 1), jnp.float32),
            jax.ShapeDtypeStruct((NPAD, 1), jnp.int32),
            jax.ShapeDtypeStruct((128, 1), jnp.int32),
        ],
    )(logits)


@functools.lru_cache(maxsize=None)
def _make_sc_gather(nrows, ncols, nidx, dtype):
    info = plsc.get_sparse_core_info()
    nw = info.num_cores * info.num_subcores
    per_w = nidx // nw
    mesh = plsc.VectorSubcoreMesh(core_axis_name="c", subcore_axis_name="s")

    @functools.partial(
        pl.kernel,
        mesh=mesh,
        out_type=jax.ShapeDtypeStruct((nidx, ncols), dtype),
        scratch_types=[
            pltpu.VMEM((per_w,), jnp.int32),
            pltpu.VMEM((per_w, ncols), dtype),
            pltpu.SemaphoreType.DMA,
        ],
    )
    def sc_gather(table_hbm, idx_hbm, out_hbm, idx_v, rows_v, sem):
        wid = lax.axis_index("s") * info.num_cores + lax.axis_index("c")
        base = wid * per_w
        pltpu.sync_copy(idx_hbm.at[pl.ds(base, per_w)], idx_v)
        pltpu.async_copy(table_hbm.at[idx_v], rows_v, sem).wait()
        pltpu.sync_copy(rows_v, out_hbm.at[pl.ds(base, per_w)])

    return sc_gather


def _sc_gather_rows(table, idx):
    return _make_sc_gather(table.shape[0], table.shape[1], idx.shape[0],
                           table.dtype)(table, idx)


def _moe_body(te_ref, x_ref, w1_ref, b1_ref, w2_ref, b2_ref, y_ref, acc_ref):
    t = pl.program_id(0)
    hc = pl.program_id(1)
    e = te_ref[t]
    oh = (lax.broadcasted_iota(jnp.int32, (1, NE), 1) == e).astype(jnp.float32)
    b1row = _dot(oh, b1_ref[0])
    h = _dot(x_ref[...].astype(jnp.bfloat16), w1_ref[0]) + b1row
    h = 0.5 * h * (1.0 + lax.erf(h * 0.7071067811865476))
    part = _dot(h.astype(jnp.bfloat16), w2_ref[0])

    @pl.when(hc == 0)
    def _():
        acc_ref[...] = part + _dot(oh, b2_ref[...])

    @pl.when((hc != 0) & (hc != HSPLIT - 1))
    def _():
        acc_ref[...] = acc_ref[...] + part

    @pl.when(hc == HSPLIT - 1)
    def _():
        y_ref[...] = acc_ref[...] + part


def _moe_call(te, xs, w1, b1, w2, b2):
    def hce(t, hc):
        return jnp.where(t % 2 == 0, hc, HSPLIT - 1 - hc)

    grid_spec = pltpu.PrefetchScalarGridSpec(
        num_scalar_prefetch=1,
        grid=(NTILES, HSPLIT),
        in_specs=[
            pl.BlockSpec((TILE, DIM), lambda t, hc, te_r: (t, 0)),
            pl.BlockSpec((1, DIM, DHC), lambda t, hc, te_r: (te_r[t], 0, hce(t, hc))),
            pl.BlockSpec((1, NE, DHC), lambda t, hc, te_r: (hce(t, hc), 0, 0)),
            pl.BlockSpec((1, DHC, DIM), lambda t, hc, te_r: (te_r[t], hce(t, hc), 0)),
            pl.BlockSpec((NE, DIM), lambda t, hc, te_r: (0, 0)),
        ],
        out_specs=pl.BlockSpec((TILE, DIM), lambda t, hc, te_r: (t, 0)),
        scratch_shapes=[pltpu.VMEM((TILE, DIM), jnp.float32)],
    )
    return pl.pallas_call(
        _moe_body,
        grid_spec=grid_spec,
        out_shape=jax.ShapeDtypeStruct((NPAD, DIM), jnp.float32),
    )(te, xs, w1, b1, w2, b2)


def _combine_body(xa_ref, gate_ref, yg_ref, o_ref):
    o_ref[...] = xa_ref[...] + gate_ref[...] * yg_ref[...]


def _combine_call(xa, gate, yg):
    blk = 256
    spec_row = pl.BlockSpec((blk, DIM), lambda i: (i, 0))
    return pl.pallas_call(
        _combine_body,
        grid=(SEQ // blk,),
        in_specs=[spec_row, pl.BlockSpec((blk, 1), lambda i: (i, 0)), spec_row],
        out_specs=spec_row,
        out_shape=jax.ShapeDtypeStruct((SEQ, DIM), jnp.float32),
    )(xa, gate, yg)


def kernel(x, gamma1, beta1, in_proj_w, in_proj_b, out_proj_w, out_proj_b,
           gamma2, beta2, Wg, bg, W1, b1, W2, b2):
    L, N, d = x.shape
    xf = x.reshape(L, d)

    q, k, v = _qkv_call(xf, gamma1.reshape(1, d), beta1.reshape(1, d),
                        in_proj_w, in_proj_b.reshape(1, 3 * d))
    qh = q.reshape(L, NH, HD).transpose(1, 0, 2)
    kh = k.reshape(L, NH, HD).transpose(1, 0, 2)
    vh = v.reshape(L, NH, HD).transpose(1, 0, 2)
    oh = _attn_call(qh, kh, vh)
    o = oh.transpose(1, 0, 2).reshape(L, d)

    wgp = jnp.concatenate([Wg, jnp.zeros((d, 128 - NE), jnp.float32)], axis=1)
    bgp = jnp.concatenate([bg, jnp.full((128 - NE,), -1e30, jnp.float32)])
    xa, x2, logits = _post_call(xf, o, out_proj_w, out_proj_b.reshape(1, d),
                                gamma2.reshape(1, d), beta2.reshape(1, d),
                                wgp, bgp.reshape(1, 128))

    dest, gate, gidx, te = _route_call(logits)

    xs = _sc_gather_rows(x2, gidx.reshape(NPAD))
    b1c = b1.reshape(NE, HSPLIT, DHC).transpose(1, 0, 2)
    ys = _moe_call(te.reshape(128)[:NTILES], xs,
                   W1.astype(jnp.bfloat16), b1c, W2.astype(jnp.bfloat16), b2)
    yg = _sc_gather_rows(ys, dest.reshape(L))

    out = _combine_call(xa, gate, yg)
    return out.reshape(L, N, d)


# speedup vs baseline: 1.2679x; 1.2679x over previous
import functools

import jax
import jax.numpy as jnp
from jax import lax
from jax.experimental import pallas as pl
from jax.experimental.pallas import tpu as pltpu
from jax.experimental.pallas import tpu_sc as plsc

DIM = 768
NH = 12
HD = 64
NE = 8
DH = 3072
SEQ = 2048
TILE = 128
NPAD = 3072
NTILES = NPAD // TILE
HSPLIT = 4
DHC = DH // HSPLIT

_DN = (((1,), (0,)), ((), ()))
_DNT = (((1,), (1,)), ((), ()))


def _dot(a, b, dn=_DN):
    return lax.dot_general(a, b, dn, preferred_element_type=jnp.float32)


def _fiota(shape, dim):
    return lax.broadcasted_iota(jnp.int32, shape, dim).astype(jnp.float32)


def _qkv_body(x_ref, g_ref, b_ref, w_ref, wb_ref, q_ref, k_ref, v_ref):
    x = x_ref[...]
    mu = jnp.mean(x, axis=1, keepdims=True)
    var = jnp.mean((x - mu) * (x - mu), axis=1, keepdims=True)
    xn = (x - mu) * lax.rsqrt(var + 1e-5) * g_ref[...] + b_ref[...]
    w = w_ref[...]
    wb = wb_ref[...]
    q_ref[...] = (_dot(xn, w[0:DIM], _DNT) + wb[0, 0:DIM]) * 0.125
    k_ref[...] = _dot(x, w[DIM:2 * DIM], _DNT) + wb[0, DIM:2 * DIM]
    v_ref[...] = _dot(x, w[2 * DIM:3 * DIM], _DNT) + wb[0, 2 * DIM:3 * DIM]


def _qkv_call(xf, g1, b1, w_in, b_in):
    blk = 256
    grid = (SEQ // blk,)
    spec_row = pl.BlockSpec((blk, DIM), lambda i: (i, 0))
    spec_full = pl.BlockSpec((DIM, DIM), lambda i: (0, 0))
    return pl.pallas_call(
        _qkv_body,
        grid=grid,
        in_specs=[
            spec_row,
            pl.BlockSpec((1, DIM), lambda i: (0, 0)),
            pl.BlockSpec((1, DIM), lambda i: (0, 0)),
            pl.BlockSpec((3 * DIM, DIM), lambda i: (0, 0)),
            pl.BlockSpec((1, 3 * DIM), lambda i: (0, 0)),
        ],
        out_specs=[spec_row, spec_row, spec_row],
        out_shape=[jax.ShapeDtypeStruct((SEQ, DIM), jnp.float32)] * 3,
    )(xf, g1, b1, w_in, b_in)


def _attn_body(q_ref, k_ref, v_ref, o_ref):
    q = q_ref[0]
    k = k_ref[0]
    v = v_ref[0]
    s = _dot(q, k, _DNT)
    m = jnp.max(s, axis=1, keepdims=True)
    p = jnp.exp(s - m)
    denom = jnp.sum(p, axis=1, keepdims=True)
    o_ref[0] = _dot(p, v) / denom


def _attn_call(qh, kh, vh):
    qblk = 512
    grid = (NH, SEQ // qblk)
    return pl.pallas_call(
        _attn_body,
        grid=grid,
        in_specs=[
            pl.BlockSpec((1, qblk, HD), lambda h, qb: (h, qb, 0)),
            pl.BlockSpec((1, SEQ, HD), lambda h, qb: (h, 0, 0)),
            pl.BlockSpec((1, SEQ, HD), lambda h, qb: (h, 0, 0)),
        ],
        out_specs=pl.BlockSpec((1, qblk, HD), lambda h, qb: (h, qb, 0)),
        out_shape=jax.ShapeDtypeStruct((NH, SEQ, HD), jnp.float32),
    )(qh, kh, vh)


def _post_body(x_ref, o_ref, wo_ref, bo_ref, g_ref, b_ref, wg_ref, bg_ref,
               xa_ref, x2_ref, lg_ref):
    xa = x_ref[...] + _dot(o_ref[...], wo_ref[...], _DNT) + bo_ref[...]
    xa_ref[...] = xa
    mu = jnp.mean(xa, axis=1, keepdims=True)
    var = jnp.mean((xa - mu) * (xa - mu), axis=1, keepdims=True)
    x2 = (xa - mu) * lax.rsqrt(var + 1e-5) * g_ref[...] + b_ref[...]
    x2_ref[...] = x2
    lg_ref[...] = _dot(x2, wg_ref[...]) + bg_ref[...]


def _post_call(xf, o, wo, bo, g2, b2, wgp, bgp):
    blk = 256
    grid = (SEQ // blk,)
    spec_row = pl.BlockSpec((blk, DIM), lambda i: (i, 0))
    return pl.pallas_call(
        _post_body,
        grid=grid,
        in_specs=[
            spec_row,
            spec_row,
            pl.BlockSpec((DIM, DIM), lambda i: (0, 0)),
            pl.BlockSpec((1, DIM), lambda i: (0, 0)),
            pl.BlockSpec((1, DIM), lambda i: (0, 0)),
            pl.BlockSpec((1, DIM), lambda i: (0, 0)),
            pl.BlockSpec((DIM, 128), lambda i: (0, 0)),
            pl.BlockSpec((1, 128), lambda i: (0, 0)),
        ],
        out_specs=[spec_row, spec_row, pl.BlockSpec((blk, 128), lambda i: (i, 0))],
        out_shape=[
            jax.ShapeDtypeStruct((SEQ, DIM), jnp.float32),
            jax.ShapeDtypeStruct((SEQ, DIM), jnp.float32),
            jax.ShapeDtypeStruct((SEQ, 128), jnp.float32),
        ],
    )(xf, o, wo, bo, g2, b2, wgp, bgp)


def _route_body(lg_ref, dest_ref, gate_ref, gidx_ref, te_ref):
    f32 = jnp.float32
    lg = lg_ref[...]
    m = jnp.max(lg, axis=1, keepdims=True)
    ex = jnp.exp(lg - m)
    sm = jnp.sum(ex, axis=1, keepdims=True)
    gate_ref[...] = 1.0 / sm
    col = _fiota((SEQ, 128), 1)
    idxf = jnp.min(jnp.where(lg == m, col, 1e9), axis=1, keepdims=True)
    onehot = (col == idxf).astype(f32)

    nblk = SEQ // TILE
    rowb = jnp.floor(_fiota((nblk, SEQ), 1) * (1.0 / TILE))
    sel = (rowb == _fiota((nblk, SEQ), 0)).astype(f32)
    bc = _dot(sel, onehot)
    tri_b = (_fiota((nblk, nblk), 0)
             > _fiota((nblk, nblk), 1)).astype(f32)
    cum_bc = _dot(tri_b, bc)
    counts = jnp.sum(bc, axis=0, keepdims=True)
    padded = jnp.floor((counts + (TILE - 1.0)) * (1.0 / TILE)) * TILE
    lt = (_fiota((128, 128), 0)
          < _fiota((128, 128), 1)).astype(f32)
    offs = _dot(padded, lt)
    ends = offs + padded

    tri_t = (_fiota((TILE, TILE), 0)
             > _fiota((TILE, TILE), 1)).astype(f32)
    eye = (_fiota((TILE, TILE), 0)
           == _fiota((TILE, TILE), 1)).astype(f32)
    dest_chunks = []
    dest_rows = []
    for c in range(nblk):
        oh_c = onehot[c * TILE:(c + 1) * TILE]
        pos_c = _dot(tri_t, oh_c) + cum_bc[c:c + 1, :]
        d_c = jnp.sum(oh_c * (offs + pos_c), axis=1, keepdims=True)
        dest_chunks.append(d_c)
        dest_rows.append(jnp.sum(eye * d_c, axis=0, keepdims=True))
    dest = jnp.concatenate(dest_chunks, axis=0)
    dest_row = jnp.concatenate(dest_rows, axis=1)
    dest_ref[...] = dest.astype(jnp.int32)

    tok_row = _fiota((1, SEQ), 1)
    gidx_chunks = []
    for c in range(NTILES):
        slot = c * TILE + _fiota((TILE, 1), 0)
        eq = (dest_row == slot).astype(f32)
        gidx_chunks.append(jnp.sum(eq * tok_row, axis=1, keepdims=True))
    gidx_ref[...] = jnp.concatenate(gidx_chunks, axis=0).astype(jnp.int32)

    lane = _fiota((1, 128), 1)
    trow = _fiota((128, 1), 0) * float(TILE)
    cmp = jnp.where((ends <= trow) & (lane < float(NE)), 1.0, 0.0)
    te = jnp.minimum(jnp.sum(cmp, axis=1, keepdims=True), float(NE - 1))
    te_ref[...] = te.astype(jnp.int32)


def _route_call(logits):
    return pl.pallas_call(
        _route_body,
        in_specs=[pl.BlockSpec((SEQ, 128), lambda: (0, 0))],
        out_specs=[
            pl.BlockSpec((SEQ, 1), lambda: (0, 0)),
            pl.BlockSpec((SEQ, 1), lambda: (0, 0)),
            pl.BlockSpec((NPAD, 1), lambda: (0, 0)),
            pl.BlockSpec((128, 1), lambda: (0, 0)),
        ],
        out_shape=[
            jax.ShapeDtypeStruct((SEQ, 1), jnp.int32),
            jax.ShapeDtypeStruct((SEQ, 1), jnp.float32),
            jax.ShapeDtypeStruct((NPAD, 1), jnp.int32),
            jax.ShapeDtypeStruct((128, 1), jnp.int32),
        ],
    )(logits)


@functools.lru_cache(maxsize=None)
def _make_sc_gather(nrows, ncols, nidx, dtype):
    info = plsc.get_sparse_core_info()
    nw = info.num_cores * info.num_subcores
    per_w = nidx // nw
    mesh = plsc.VectorSubcoreMesh(core_axis_name="c", subcore_axis_name="s")

    @functools.partial(
        pl.kernel,
        mesh=mesh,
        out_type=jax.ShapeDtypeStruct((nidx, ncols), dtype),
        scratch_types=[
            pltpu.VMEM((per_w,), jnp.int32),
            pltpu.VMEM((per_w, ncols), dtype),
            pltpu.SemaphoreType.DMA,
        ],
    )
    def sc_gather(table_hbm, idx_hbm, out_hbm, idx_v, rows_v, sem):
        wid = lax.axis_index("s") * info.num_cores + lax.axis_index("c")
        base = wid * per_w
        pltpu.sync_copy(idx_hbm.at[pl.ds(base, per_w)], idx_v)
        pltpu.async_copy(table_hbm.at[idx_v], rows_v, sem).wait()
        pltpu.sync_copy(rows_v, out_hbm.at[pl.ds(base, per_w)])

    return sc_gather


def _sc_gather_rows(table, idx):
    return _make_sc_gather(table.shape[0], table.shape[1], idx.shape[0],
                           table.dtype)(table, idx)


def _moe_body(te_ref, x_ref, w1_ref, b1_ref, w2_ref, b2_ref, y_ref):
    t = pl.program_id(0)
    e = te_ref[t]
    oh = (lax.broadcasted_iota(jnp.int32, (1, NE), 1) == e).astype(jnp.float32)
    h = _dot(x_ref[...], w1_ref[0]) + _dot(oh, b1_ref[...])
    h = 0.5 * h * (1.0 + lax.erf(h * 0.7071067811865476))
    y_ref[...] = _dot(h, w2_ref[0]) + _dot(oh, b2_ref[...])


def _moe_call(te, xs, w1, b1, w2, b2):
    grid_spec = pltpu.PrefetchScalarGridSpec(
        num_scalar_prefetch=1,
        grid=(NTILES,),
        in_specs=[
            pl.BlockSpec((TILE, DIM), lambda t, te_r: (t, 0)),
            pl.BlockSpec((1, DIM, DH), lambda t, te_r: (te_r[t], 0, 0)),
            pl.BlockSpec((NE, DH), lambda t, te_r: (0, 0)),
            pl.BlockSpec((1, DH, DIM), lambda t, te_r: (te_r[t], 0, 0)),
            pl.BlockSpec((NE, DIM), lambda t, te_r: (0, 0)),
        ],
        out_specs=pl.BlockSpec((TILE, DIM), lambda t, te_r: (t, 0)),
    )
    return pl.pallas_call(
        _moe_body,
        grid_spec=grid_spec,
        out_shape=jax.ShapeDtypeStruct((NPAD, DIM), jnp.float32),
    )(te, xs, w1, b1, w2, b2)


def _combine_body(xa_ref, gate_ref, yg_ref, o_ref):
    o_ref[...] = xa_ref[...] + gate_ref[...] * yg_ref[...]


def _combine_call(xa, gate, yg):
    blk = 256
    spec_row = pl.BlockSpec((blk, DIM), lambda i: (i, 0))
    return pl.pallas_call(
        _combine_body,
        grid=(SEQ // blk,),
        in_specs=[spec_row, pl.BlockSpec((blk, 1), lambda i: (i, 0)), spec_row],
        out_specs=spec_row,
        out_shape=jax.ShapeDtypeStruct((SEQ, DIM), jnp.float32),
    )(xa, gate, yg)


def kernel(x, gamma1, beta1, in_proj_w, in_proj_b, out_proj_w, out_proj_b,
           gamma2, beta2, Wg, bg, W1, b1, W2, b2):
    L, N, d = x.shape
    xf = x.reshape(L, d)

    q, k, v = _qkv_call(xf, gamma1.reshape(1, d), beta1.reshape(1, d),
                        in_proj_w, in_proj_b.reshape(1, 3 * d))
    qh = q.reshape(L, NH, HD).transpose(1, 0, 2)
    kh = k.reshape(L, NH, HD).transpose(1, 0, 2)
    vh = v.reshape(L, NH, HD).transpose(1, 0, 2)
    oh = _attn_call(qh, kh, vh)
    o = oh.transpose(1, 0, 2).reshape(L, d)

    wgp = jnp.concatenate([Wg, jnp.zeros((d, 128 - NE), jnp.float32)], axis=1)
    bgp = jnp.concatenate([bg, jnp.full((128 - NE,), -1e30, jnp.float32)])
    xa, x2, logits = _post_call(xf, o, out_proj_w, out_proj_b.reshape(1, d),
                                gamma2.reshape(1, d), beta2.reshape(1, d),
                                wgp, bgp.reshape(1, 128))

    dest, gate, gidx, te = _route_call(logits)

    xs = _sc_gather_rows(x2, gidx.reshape(NPAD))
    ys = _moe_call(te.reshape(128)[:NTILES], xs, W1, b1, W2, b2)
    yg = _sc_gather_rows(ys, dest.reshape(L))

    out = _combine_call(xa, gate, yg)
    return out.reshape(L, N, d)


# speedup vs baseline: 1.2723x; 1.0035x over previous
import functools

import jax
import jax.numpy as jnp
from jax import lax
from jax.experimental import pallas as pl
from jax.experimental.pallas import tpu as pltpu
from jax.experimental.pallas import tpu_sc as plsc

DIM = 768
NH = 12
HD = 64
NE = 8
DH = 3072
SEQ = 2048
TILE = 128
NPAD = 3072
NTILES = NPAD // TILE
HSPLIT = 4
DHC = DH // HSPLIT

_DN = (((1,), (0,)), ((), ()))
_DNT = (((1,), (1,)), ((), ()))


def _dot(a, b, dn=_DN):
    return lax.dot_general(a, b, dn, preferred_element_type=jnp.float32)


def _fiota(shape, dim):
    return lax.broadcasted_iota(jnp.int32, shape, dim).astype(jnp.float32)


def _qkv_body(x_ref, g_ref, b_ref, w_ref, wb_ref, q_ref, k_ref, v_ref):
    x = x_ref[...]
    mu = jnp.mean(x, axis=1, keepdims=True)
    var = jnp.mean((x - mu) * (x - mu), axis=1, keepdims=True)
    xn = (x - mu) * lax.rsqrt(var + 1e-5) * g_ref[...] + b_ref[...]
    w = w_ref[...]
    wb = wb_ref[...]
    q_ref[...] = (_dot(xn, w[0:DIM], _DNT) + wb[0, 0:DIM]) * 0.125
    k_ref[...] = _dot(x, w[DIM:2 * DIM], _DNT) + wb[0, DIM:2 * DIM]
    v_ref[...] = _dot(x, w[2 * DIM:3 * DIM], _DNT) + wb[0, 2 * DIM:3 * DIM]


def _qkv_call(xf, g1, b1, w_in, b_in):
    blk = 256
    grid = (SEQ // blk,)
    spec_row = pl.BlockSpec((blk, DIM), lambda i: (i, 0))
    spec_full = pl.BlockSpec((DIM, DIM), lambda i: (0, 0))
    return pl.pallas_call(
        _qkv_body,
        grid=grid,
        in_specs=[
            spec_row,
            pl.BlockSpec((1, DIM), lambda i: (0, 0)),
            pl.BlockSpec((1, DIM), lambda i: (0, 0)),
            pl.BlockSpec((3 * DIM, DIM), lambda i: (0, 0)),
            pl.BlockSpec((1, 3 * DIM), lambda i: (0, 0)),
        ],
        out_specs=[spec_row, spec_row, spec_row],
        out_shape=[jax.ShapeDtypeStruct((SEQ, DIM), jnp.float32)] * 3,
    )(xf, g1, b1, w_in, b_in)


def _attn_body(q_ref, k_ref, v_ref, o_ref):
    q = q_ref[0]
    k = k_ref[0]
    v = v_ref[0]
    s = _dot(q, k, _DNT)
    m = jnp.max(s, axis=1, keepdims=True)
    p = jnp.exp(s - m)
    denom = jnp.sum(p, axis=1, keepdims=True)
    o_ref[0] = _dot(p, v) / denom


def _attn_call(qh, kh, vh):
    qblk = 512
    grid = (NH, SEQ // qblk)
    return pl.pallas_call(
        _attn_body,
        grid=grid,
        in_specs=[
            pl.BlockSpec((1, qblk, HD), lambda h, qb: (h, qb, 0)),
            pl.BlockSpec((1, SEQ, HD), lambda h, qb: (h, 0, 0)),
            pl.BlockSpec((1, SEQ, HD), lambda h, qb: (h, 0, 0)),
        ],
        out_specs=pl.BlockSpec((1, qblk, HD), lambda h, qb: (h, qb, 0)),
        out_shape=jax.ShapeDtypeStruct((NH, SEQ, HD), jnp.float32),
    )(qh, kh, vh)


def _post_body(x_ref, o_ref, wo_ref, bo_ref, g_ref, b_ref, wg_ref, bg_ref,
               xa_ref, x2_ref, lg_ref):
    xa = x_ref[...] + _dot(o_ref[...], wo_ref[...], _DNT) + bo_ref[...]
    xa_ref[...] = xa
    mu = jnp.mean(xa, axis=1, keepdims=True)
    var = jnp.mean((xa - mu) * (xa - mu), axis=1, keepdims=True)
    x2 = (xa - mu) * lax.rsqrt(var + 1e-5) * g_ref[...] + b_ref[...]
    x2_ref[...] = x2
    lg_ref[...] = _dot(x2, wg_ref[...]) + bg_ref[...]


def _post_call(xf, o, wo, bo, g2, b2, wgp, bgp):
    blk = 256
    grid = (SEQ // blk,)
    spec_row = pl.BlockSpec((blk, DIM), lambda i: (i, 0))
    return pl.pallas_call(
        _post_body,
        grid=grid,
        in_specs=[
            spec_row,
            spec_row,
            pl.BlockSpec((DIM, DIM), lambda i: (0, 0)),
            pl.BlockSpec((1, DIM), lambda i: (0, 0)),
            pl.BlockSpec((1, DIM), lambda i: (0, 0)),
            pl.BlockSpec((1, DIM), lambda i: (0, 0)),
            pl.BlockSpec((DIM, 128), lambda i: (0, 0)),
            pl.BlockSpec((1, 128), lambda i: (0, 0)),
        ],
        out_specs=[spec_row, spec_row, pl.BlockSpec((blk, 128), lambda i: (i, 0))],
        out_shape=[
            jax.ShapeDtypeStruct((SEQ, DIM), jnp.float32),
            jax.ShapeDtypeStruct((SEQ, DIM), jnp.float32),
            jax.ShapeDtypeStruct((SEQ, 128), jnp.float32),
        ],
    )(xf, o, wo, bo, g2, b2, wgp, bgp)


def _route_body(lg_ref, dest_ref, gate_ref, gidx_ref, te_ref):
    f32 = jnp.float32
    lg = lg_ref[...]
    m = jnp.max(lg, axis=1, keepdims=True)
    ex = jnp.exp(lg - m)
    sm = jnp.sum(ex, axis=1, keepdims=True)
    gate_ref[...] = 1.0 / sm
    col = _fiota((SEQ, 128), 1)
    idxf = jnp.min(jnp.where(lg == m, col, 1e9), axis=1, keepdims=True)
    onehot = (col == idxf).astype(f32)

    nblk = SEQ // TILE
    rowb = jnp.floor(_fiota((nblk, SEQ), 1) * (1.0 / TILE))
    sel = (rowb == _fiota((nblk, SEQ), 0)).astype(f32)
    bc = _dot(sel, onehot)
    tri_b = (_fiota((nblk, nblk), 0)
             > _fiota((nblk, nblk), 1)).astype(f32)
    cum_bc = _dot(tri_b, bc)
    counts = jnp.sum(bc, axis=0, keepdims=True)
    padded = jnp.floor((counts + (TILE - 1.0)) * (1.0 / TILE)) * TILE
    lt = (_fiota((128, 128), 0)
          < _fiota((128, 128), 1)).astype(f32)
    offs = _dot(padded, lt)
    ends = offs + padded

    tri_t = (_fiota((TILE, TILE), 0)
             > _fiota((TILE, TILE), 1)).astype(f32)
    eye = (_fiota((TILE, TILE), 0)
           == _fiota((TILE, TILE), 1)).astype(f32)
    dest_chunks = []
    dest_rows = []
    for c in range(nblk):
        oh_c = onehot[c * TILE:(c + 1) * TILE]
        pos_c = _dot(tri_t, oh_c) + cum_bc[c:c + 1, :]
        d_c = jnp.sum(oh_c * (offs + pos_c), axis=1, keepdims=True)
        dest_chunks.append(d_c)
        dest_rows.append(jnp.sum(eye * d_c, axis=0, keepdims=True))
    dest = jnp.concatenate(dest_chunks, axis=0)
    dest_row = jnp.concatenate(dest_rows, axis=1)
    dest_ref[...] = dest.astype(jnp.int32)

    tok_row = _fiota((1, SEQ), 1)
    gidx_chunks = []
    for c in range(NTILES):
        slot = c * TILE + _fiota((TILE, 1), 0)
        eq = (dest_row == slot).astype(f32)
        gidx_chunks.append(jnp.sum(eq * tok_row, axis=1, keepdims=True))
    gidx_ref[...] = jnp.concatenate(gidx_chunks, axis=0).astype(jnp.int32)

    lane = _fiota((1, 128), 1)
    trow = _fiota((128, 1), 0) * float(TILE)
    cmp = jnp.where((ends <= trow) & (lane < float(NE)), 1.0, 0.0)
    te = jnp.minimum(jnp.sum(cmp, axis=1, keepdims=True), float(NE - 1))
    te_ref[...] = te.astype(jnp.int32)


def _route_call(logits):
    return pl.pallas_call(
        _route_body,
        in_specs=[pl.BlockSpec((SEQ, 128), lambda: (0, 0))],
        out_specs=[
            pl.BlockSpec((SEQ, 1), lambda: (0, 0)),
            pl.BlockSpec((SEQ, 1), lambda: (0, 0)),
            pl.BlockSpec((NPAD, 1), lambda: (0, 0)),
            pl.BlockSpec((128, 1), lambda: (0, 0)),
        ],
        out_shape=[
            jax.ShapeDtypeStruct((SEQ, 1), jnp.int32),
            jax.ShapeDtypeStruct((SEQ, 1), jnp.float32),
            jax.ShapeDtypeStruct((NPAD, 1), jnp.int32),
            jax.ShapeDtypeStruct((128, 1), jnp.int32),
        ],
    )(logits)


@functools.lru_cache(maxsize=None)
def _make_sc_gather(nrows, ncols, nidx, dtype):
    info = plsc.get_sparse_core_info()
    nw = info.num_cores * info.num_subcores
    per_w = nidx // nw
    mesh = plsc.VectorSubcoreMesh(core_axis_name="c", subcore_axis_name="s")

    half = per_w // 2

    @functools.partial(
        pl.kernel,
        mesh=mesh,
        out_type=jax.ShapeDtypeStruct((nidx, ncols), dtype),
        scratch_types=[
            pltpu.VMEM((per_w,), jnp.int32),
            pltpu.VMEM((half, ncols), dtype),
            pltpu.VMEM((half, ncols), dtype),
            pltpu.SemaphoreType.DMA,
            pltpu.SemaphoreType.DMA,
            pltpu.SemaphoreType.DMA,
            pltpu.SemaphoreType.DMA,
        ],
    )
    def sc_gather(table_hbm, idx_hbm, out_hbm, idx_v, buf0, buf1,
                  g0, g1, s0, s1):
        wid = lax.axis_index("s") * info.num_cores + lax.axis_index("c")
        base = wid * per_w
        pltpu.sync_copy(idx_hbm.at[pl.ds(base, per_w)], idx_v)
        c0 = pltpu.async_copy(table_hbm.at[idx_v.at[pl.ds(0, half)]], buf0, g0)
        c1 = pltpu.async_copy(table_hbm.at[idx_v.at[pl.ds(half, half)]], buf1, g1)
        c0.wait()
        o0 = pltpu.async_copy(buf0, out_hbm.at[pl.ds(base, half)], s0)
        c1.wait()
        o1 = pltpu.async_copy(buf1, out_hbm.at[pl.ds(base + half, half)], s1)
        o0.wait()
        o1.wait()

    return sc_gather


def _sc_gather_rows(table, idx):
    return _make_sc_gather(table.shape[0], table.shape[1], idx.shape[0],
                           table.dtype)(table, idx)


def _moe_body(te_ref, x_ref, w1_ref, b1_ref, w2_ref, b2_ref, y_ref):
    t = pl.program_id(0)
    e = te_ref[t]
    oh = (lax.broadcasted_iota(jnp.int32, (1, NE), 1) == e).astype(jnp.float32)
    h = _dot(x_ref[...], w1_ref[0]) + _dot(oh, b1_ref[...])
    h = 0.5 * h * (1.0 + lax.erf(h * 0.7071067811865476))
    y_ref[...] = _dot(h, w2_ref[0]) + _dot(oh, b2_ref[...])


def _moe_call(te, xs, w1, b1, w2, b2):
    grid_spec = pltpu.PrefetchScalarGridSpec(
        num_scalar_prefetch=1,
        grid=(NTILES,),
        in_specs=[
            pl.BlockSpec((TILE, DIM), lambda t, te_r: (t, 0)),
            pl.BlockSpec((1, DIM, DH), lambda t, te_r: (te_r[t], 0, 0)),
            pl.BlockSpec((NE, DH), lambda t, te_r: (0, 0)),
            pl.BlockSpec((1, DH, DIM), lambda t, te_r: (te_r[t], 0, 0)),
            pl.BlockSpec((NE, DIM), lambda t, te_r: (0, 0)),
        ],
        out_specs=pl.BlockSpec((TILE, DIM), lambda t, te_r: (t, 0)),
    )
    return pl.pallas_call(
        _moe_body,
        grid_spec=grid_spec,
        out_shape=jax.ShapeDtypeStruct((NPAD, DIM), jnp.float32),
    )(te, xs, w1, b1, w2, b2)


def _combine_body(xa_ref, gate_ref, yg_ref, o_ref):
    o_ref[...] = xa_ref[...] + gate_ref[...] * yg_ref[...]


def _combine_call(xa, gate, yg):
    blk = 256
    spec_row = pl.BlockSpec((blk, DIM), lambda i: (i, 0))
    return pl.pallas_call(
        _combine_body,
        grid=(SEQ // blk,),
        in_specs=[spec_row, pl.BlockSpec((blk, 1), lambda i: (i, 0)), spec_row],
        out_specs=spec_row,
        out_shape=jax.ShapeDtypeStruct((SEQ, DIM), jnp.float32),
    )(xa, gate, yg)


def kernel(x, gamma1, beta1, in_proj_w, in_proj_b, out_proj_w, out_proj_b,
           gamma2, beta2, Wg, bg, W1, b1, W2, b2):
    L, N, d = x.shape
    xf = x.reshape(L, d)

    q, k, v = _qkv_call(xf, gamma1.reshape(1, d), beta1.reshape(1, d),
                        in_proj_w, in_proj_b.reshape(1, 3 * d))
    qh = q.reshape(L, NH, HD).transpose(1, 0, 2)
    kh = k.reshape(L, NH, HD).transpose(1, 0, 2)
    vh = v.reshape(L, NH, HD).transpose(1, 0, 2)
    oh = _attn_call(qh, kh, vh)
    o = oh.transpose(1, 0, 2).reshape(L, d)

    wgp = jnp.concatenate([Wg, jnp.zeros((d, 128 - NE), jnp.float32)], axis=1)
    bgp = jnp.concatenate([bg, jnp.full((128 - NE,), -1e30, jnp.float32)])
    xa, x2, logits = _post_call(xf, o, out_proj_w, out_proj_b.reshape(1, d),
                                gamma2.reshape(1, d), beta2.reshape(1, d),
                                wgp, bgp.reshape(1, 128))

    dest, gate, gidx, te = _route_call(logits)

    xs = _sc_gather_rows(x2, gidx.reshape(NPAD))
    ys = _moe_call(te.reshape(128)[:NTILES], xs, W1, b1, W2, b2)
    yg = _sc_gather_rows(ys, dest.reshape(L))

    out = _combine_call(xa, gate, yg)
    return out.reshape(L, N, d)


# speedup vs baseline: 1.6025x; 1.2595x over previous
import functools

import jax
import jax.numpy as jnp
from jax import lax
from jax.experimental import pallas as pl
from jax.experimental.pallas import tpu as pltpu
from jax.experimental.pallas import tpu_sc as plsc

DIM = 768
NH = 12
HD = 64
NE = 8
DH = 3072
SEQ = 2048
TILE = 128
NPAD = 3072
NTILES = NPAD // TILE
HSPLIT = 4
DHC = DH // HSPLIT

_DN = (((1,), (0,)), ((), ()))
_DNT = (((1,), (1,)), ((), ()))


def _dot(a, b, dn=_DN):
    return lax.dot_general(a, b, dn, preferred_element_type=jnp.float32)


def _fiota(shape, dim):
    return lax.broadcasted_iota(jnp.int32, shape, dim).astype(jnp.float32)


def _qkv_body(x_ref, g_ref, b_ref, w_ref, wb_ref, q_ref, k_ref, v_ref):
    x = x_ref[...]
    mu = jnp.mean(x, axis=1, keepdims=True)
    var = jnp.mean((x - mu) * (x - mu), axis=1, keepdims=True)
    xn = (x - mu) * lax.rsqrt(var + 1e-5) * g_ref[...] + b_ref[...]
    w = w_ref[...]
    wb = wb_ref[...]
    q_ref[...] = (_dot(xn, w[0:DIM], _DNT) + wb[0, 0:DIM]) * 0.125
    k_ref[...] = _dot(x, w[DIM:2 * DIM], _DNT) + wb[0, DIM:2 * DIM]
    v_ref[...] = _dot(x, w[2 * DIM:3 * DIM], _DNT) + wb[0, 2 * DIM:3 * DIM]


def _qkv_call(xf, g1, b1, w_in, b_in):
    blk = 256
    grid = (SEQ // blk,)
    spec_row = pl.BlockSpec((blk, DIM), lambda i: (i, 0))
    spec_full = pl.BlockSpec((DIM, DIM), lambda i: (0, 0))
    return pl.pallas_call(
        _qkv_body,
        grid=grid,
        in_specs=[
            spec_row,
            pl.BlockSpec((1, DIM), lambda i: (0, 0)),
            pl.BlockSpec((1, DIM), lambda i: (0, 0)),
            pl.BlockSpec((3 * DIM, DIM), lambda i: (0, 0)),
            pl.BlockSpec((1, 3 * DIM), lambda i: (0, 0)),
        ],
        out_specs=[spec_row, spec_row, spec_row],
        out_shape=[jax.ShapeDtypeStruct((SEQ, DIM), jnp.float32)] * 3,
    )(xf, g1, b1, w_in, b_in)


def _attn_body(q_ref, k_ref, v_ref, o_ref):
    for u in range(2):
        q = q_ref[:, u * HD:(u + 1) * HD]
        k = k_ref[:, u * HD:(u + 1) * HD]
        v = v_ref[:, u * HD:(u + 1) * HD]
        s = _dot(q, k, _DNT)
        m = jnp.max(s, axis=1, keepdims=True)
        p = jnp.exp(s - m)
        denom = jnp.sum(p, axis=1, keepdims=True)
        o_ref[:, u * HD:(u + 1) * HD] = _dot(p, v) / denom


def _attn_call(q, k, v):
    qblk = 512
    grid = (NH // 2, SEQ // qblk)
    return pl.pallas_call(
        _attn_body,
        grid=grid,
        in_specs=[
            pl.BlockSpec((qblk, 2 * HD), lambda h2, qb: (qb, h2)),
            pl.BlockSpec((SEQ, 2 * HD), lambda h2, qb: (0, h2)),
            pl.BlockSpec((SEQ, 2 * HD), lambda h2, qb: (0, h2)),
        ],
        out_specs=pl.BlockSpec((qblk, 2 * HD), lambda h2, qb: (qb, h2)),
        out_shape=jax.ShapeDtypeStruct((SEQ, DIM), jnp.float32),
    )(q, k, v)


def _post_body(x_ref, o_ref, wo_ref, bo_ref, g_ref, b_ref, wg_ref, bg_ref,
               xa_ref, x2_ref, lg_ref):
    xa = x_ref[...] + _dot(o_ref[...], wo_ref[...], _DNT) + bo_ref[...]
    xa_ref[...] = xa
    mu = jnp.mean(xa, axis=1, keepdims=True)
    var = jnp.mean((xa - mu) * (xa - mu), axis=1, keepdims=True)
    x2 = (xa - mu) * lax.rsqrt(var + 1e-5) * g_ref[...] + b_ref[...]
    x2_ref[...] = x2
    lg_ref[...] = _dot(x2, wg_ref[...]) + bg_ref[...]


def _post_call(xf, o, wo, bo, g2, b2, wgp, bgp):
    blk = 256
    grid = (SEQ // blk,)
    spec_row = pl.BlockSpec((blk, DIM), lambda i: (i, 0))
    return pl.pallas_call(
        _post_body,
        grid=grid,
        in_specs=[
            spec_row,
            spec_row,
            pl.BlockSpec((DIM, DIM), lambda i: (0, 0)),
            pl.BlockSpec((1, DIM), lambda i: (0, 0)),
            pl.BlockSpec((1, DIM), lambda i: (0, 0)),
            pl.BlockSpec((1, DIM), lambda i: (0, 0)),
            pl.BlockSpec((DIM, 128), lambda i: (0, 0)),
            pl.BlockSpec((1, 128), lambda i: (0, 0)),
        ],
        out_specs=[spec_row, spec_row, pl.BlockSpec((blk, 128), lambda i: (i, 0))],
        out_shape=[
            jax.ShapeDtypeStruct((SEQ, DIM), jnp.float32),
            jax.ShapeDtypeStruct((SEQ, DIM), jnp.float32),
            jax.ShapeDtypeStruct((SEQ, 128), jnp.float32),
        ],
    )(xf, o, wo, bo, g2, b2, wgp, bgp)


def _route_body(lg_ref, dest_ref, gate_ref, gidx_ref, te_ref):
    f32 = jnp.float32
    lg = lg_ref[...]
    m = jnp.max(lg, axis=1, keepdims=True)
    ex = jnp.exp(lg - m)
    sm = jnp.sum(ex, axis=1, keepdims=True)
    gate_ref[...] = 1.0 / sm
    col = _fiota((SEQ, 128), 1)
    idxf = jnp.min(jnp.where(lg == m, col, 1e9), axis=1, keepdims=True)
    onehot = (col == idxf).astype(f32)

    nblk = SEQ // TILE
    rowb = jnp.floor(_fiota((nblk, SEQ), 1) * (1.0 / TILE))
    sel = (rowb == _fiota((nblk, SEQ), 0)).astype(f32)
    bc = _dot(sel, onehot)
    tri_b = (_fiota((nblk, nblk), 0)
             > _fiota((nblk, nblk), 1)).astype(f32)
    cum_bc = _dot(tri_b, bc)
    counts = jnp.sum(bc, axis=0, keepdims=True)
    padded = jnp.floor((counts + (TILE - 1.0)) * (1.0 / TILE)) * TILE
    lt = (_fiota((128, 128), 0)
          < _fiota((128, 128), 1)).astype(f32)
    offs = _dot(padded, lt)
    ends = offs + padded

    tri_t = (_fiota((TILE, TILE), 0)
             > _fiota((TILE, TILE), 1)).astype(f32)
    eye = (_fiota((TILE, TILE), 0)
           == _fiota((TILE, TILE), 1)).astype(f32)
    dest_chunks = []
    dest_rows = []
    for c in range(nblk):
        oh_c = onehot[c * TILE:(c + 1) * TILE]
        pos_c = _dot(tri_t, oh_c) + cum_bc[c:c + 1, :]
        d_c = jnp.sum(oh_c * (offs + pos_c), axis=1, keepdims=True)
        dest_chunks.append(d_c)
        dest_rows.append(jnp.sum(eye * d_c, axis=0, keepdims=True))
    dest = jnp.concatenate(dest_chunks, axis=0)
    dest_row = jnp.concatenate(dest_rows, axis=1)
    dest_ref[...] = dest.astype(jnp.int32)

    tok_row = _fiota((1, SEQ), 1)
    gidx_chunks = []
    for c in range(NTILES):
        slot = c * TILE + _fiota((TILE, 1), 0)
        eq = (dest_row == slot).astype(f32)
        gidx_chunks.append(jnp.sum(eq * tok_row, axis=1, keepdims=True))
    gidx_ref[...] = jnp.concatenate(gidx_chunks, axis=0).astype(jnp.int32)

    lane = _fiota((1, 128), 1)
    trow = _fiota((128, 1), 0) * float(TILE)
    cmp = jnp.where((ends <= trow) & (lane < float(NE)), 1.0, 0.0)
    te = jnp.minimum(jnp.sum(cmp, axis=1, keepdims=True), float(NE - 1))
    te_ref[...] = te.astype(jnp.int32)


def _route_call(logits):
    return pl.pallas_call(
        _route_body,
        in_specs=[pl.BlockSpec((SEQ, 128), lambda: (0, 0))],
        out_specs=[
            pl.BlockSpec((SEQ, 1), lambda: (0, 0)),
            pl.BlockSpec((SEQ, 1), lambda: (0, 0)),
            pl.BlockSpec((NPAD, 1), lambda: (0, 0)),
            pl.BlockSpec((128, 1), lambda: (0, 0)),
        ],
        out_shape=[
            jax.ShapeDtypeStruct((SEQ, 1), jnp.int32),
            jax.ShapeDtypeStruct((SEQ, 1), jnp.float32),
            jax.ShapeDtypeStruct((NPAD, 1), jnp.int32),
            jax.ShapeDtypeStruct((128, 1), jnp.int32),
        ],
    )(logits)


@functools.lru_cache(maxsize=None)
def _make_sc_gather(nrows, ncols, nidx, dtype):
    info = plsc.get_sparse_core_info()
    nw = info.num_cores * info.num_subcores
    per_w = nidx // nw
    mesh = plsc.VectorSubcoreMesh(core_axis_name="c", subcore_axis_name="s")

    @functools.partial(
        pl.kernel,
        mesh=mesh,
        out_type=jax.ShapeDtypeStruct((nidx, ncols), dtype),
        scratch_types=[
            pltpu.VMEM((per_w,), jnp.int32),
            pltpu.VMEM((per_w, ncols), dtype),
            pltpu.SemaphoreType.DMA,
        ],
    )
    def sc_gather(table_hbm, idx_hbm, out_hbm, idx_v, rows_v, sem):
        wid = lax.axis_index("s") * info.num_cores + lax.axis_index("c")
        base = wid * per_w
        pltpu.sync_copy(idx_hbm.at[pl.ds(base, per_w)], idx_v)
        pltpu.async_copy(table_hbm.at[idx_v], rows_v, sem).wait()
        pltpu.sync_copy(rows_v, out_hbm.at[pl.ds(base, per_w)])

    return sc_gather


def _sc_gather_rows(table, idx):
    return _make_sc_gather(table.shape[0], table.shape[1], idx.shape[0],
                           table.dtype)(table, idx)


def _moe_body(te_ref, x_ref, w1_ref, b1_ref, w2_ref, b2_ref, y_ref):
    t = pl.program_id(0)
    e = te_ref[t]
    oh = (lax.broadcasted_iota(jnp.int32, (1, NE), 1) == e).astype(jnp.float32)
    h = _dot(x_ref[...], w1_ref[0]) + _dot(oh, b1_ref[...])
    h = 0.5 * h * (1.0 + lax.erf(h * 0.7071067811865476))
    y_ref[...] = _dot(h, w2_ref[0]) + _dot(oh, b2_ref[...])


def _moe_call(te, xs, w1, b1, w2, b2):
    grid_spec = pltpu.PrefetchScalarGridSpec(
        num_scalar_prefetch=1,
        grid=(NTILES,),
        in_specs=[
            pl.BlockSpec((TILE, DIM), lambda t, te_r: (t, 0)),
            pl.BlockSpec((1, DIM, DH), lambda t, te_r: (te_r[t], 0, 0)),
            pl.BlockSpec((NE, DH), lambda t, te_r: (0, 0)),
            pl.BlockSpec((1, DH, DIM), lambda t, te_r: (te_r[t], 0, 0)),
            pl.BlockSpec((NE, DIM), lambda t, te_r: (0, 0)),
        ],
        out_specs=pl.BlockSpec((TILE, DIM), lambda t, te_r: (t, 0)),
    )
    return pl.pallas_call(
        _moe_body,
        grid_spec=grid_spec,
        out_shape=jax.ShapeDtypeStruct((NPAD, DIM), jnp.float32),
    )(te, xs, w1, b1, w2, b2)


def _combine_body(xa_ref, gate_ref, yg_ref, o_ref):
    o_ref[...] = xa_ref[...] + gate_ref[...] * yg_ref[...]


def _combine_call(xa, gate, yg):
    blk = 256
    spec_row = pl.BlockSpec((blk, DIM), lambda i: (i, 0))
    return pl.pallas_call(
        _combine_body,
        grid=(SEQ // blk,),
        in_specs=[spec_row, pl.BlockSpec((blk, 1), lambda i: (i, 0)), spec_row],
        out_specs=spec_row,
        out_shape=jax.ShapeDtypeStruct((SEQ, DIM), jnp.float32),
    )(xa, gate, yg)


def kernel(x, gamma1, beta1, in_proj_w, in_proj_b, out_proj_w, out_proj_b,
           gamma2, beta2, Wg, bg, W1, b1, W2, b2):
    L, N, d = x.shape
    xf = x.reshape(L, d)

    q, k, v = _qkv_call(xf, gamma1.reshape(1, d), beta1.reshape(1, d),
                        in_proj_w, in_proj_b.reshape(1, 3 * d))
    o = _attn_call(q, k, v)

    wgp = jnp.concatenate([Wg, jnp.zeros((d, 128 - NE), jnp.float32)], axis=1)
    bgp = jnp.concatenate([bg, jnp.full((128 - NE,), -1e30, jnp.float32)])
    xa, x2, logits = _post_call(xf, o, out_proj_w, out_proj_b.reshape(1, d),
                                gamma2.reshape(1, d), beta2.reshape(1, d),
                                wgp, bgp.reshape(1, 128))

    dest, gate, gidx, te = _route_call(logits)

    xs = _sc_gather_rows(x2, gidx.reshape(NPAD))
    ys = _moe_call(te.reshape(128)[:NTILES], xs, W1, b1, W2, b2)
    yg = _sc_gather_rows(ys, dest.reshape(L))

    out = _combine_call(xa, gate, yg)
    return out.reshape(L, N, d)


# speedup vs baseline: 1.8925x; 1.1810x over previous
import functools

import jax
import jax.numpy as jnp
from jax import lax
from jax.experimental import pallas as pl
from jax.experimental.pallas import tpu as pltpu
from jax.experimental.pallas import tpu_sc as plsc

DIM = 768
NH = 12
HD = 64
NE = 8
DH = 3072
SEQ = 2048
TILE = 128
NPAD = 3072
NTILES = NPAD // TILE
HSPLIT = 4
DHC = DH // HSPLIT

_DN = (((1,), (0,)), ((), ()))
_DNT = (((1,), (1,)), ((), ()))


def _dot(a, b, dn=_DN):
    return lax.dot_general(a, b, dn, preferred_element_type=jnp.float32)


def _fiota(shape, dim):
    return lax.broadcasted_iota(jnp.int32, shape, dim).astype(jnp.float32)


def _qkv_body(x_ref, g_ref, b_ref, w_ref, wb_ref, q_ref, k_ref, v_ref):
    x = x_ref[...]
    mu = jnp.mean(x, axis=1, keepdims=True)
    var = jnp.mean((x - mu) * (x - mu), axis=1, keepdims=True)
    xn = (x - mu) * lax.rsqrt(var + 1e-5) * g_ref[...] + b_ref[...]
    w = w_ref[...]
    wb = wb_ref[...]
    q_ref[...] = (_dot(xn, w[0:DIM], _DNT) + wb[0, 0:DIM]) * 0.125
    k_ref[...] = _dot(x, w[DIM:2 * DIM], _DNT) + wb[0, DIM:2 * DIM]
    v_ref[...] = _dot(x, w[2 * DIM:3 * DIM], _DNT) + wb[0, 2 * DIM:3 * DIM]


def _qkv_call(xf, g1, b1, w_in, b_in):
    blk = 256
    grid = (SEQ // blk,)
    spec_row = pl.BlockSpec((blk, DIM), lambda i: (i, 0))
    spec_full = pl.BlockSpec((DIM, DIM), lambda i: (0, 0))
    return pl.pallas_call(
        _qkv_body,
        grid=grid,
        in_specs=[
            spec_row,
            pl.BlockSpec((1, DIM), lambda i: (0, 0)),
            pl.BlockSpec((1, DIM), lambda i: (0, 0)),
            pl.BlockSpec((3 * DIM, DIM), lambda i: (0, 0)),
            pl.BlockSpec((1, 3 * DIM), lambda i: (0, 0)),
        ],
        out_specs=[spec_row, spec_row, spec_row],
        out_shape=[jax.ShapeDtypeStruct((SEQ, DIM), jnp.float32)] * 3,
    )(xf, g1, b1, w_in, b_in)


def _attn_body(q_ref, k_ref, v_ref, o_ref):
    for u in range(2):
        q = q_ref[:, u * HD:(u + 1) * HD]
        k = k_ref[:, u * HD:(u + 1) * HD]
        v = v_ref[:, u * HD:(u + 1) * HD]
        s = _dot(q, k, _DNT)
        m = jnp.max(s, axis=1, keepdims=True)
        p = jnp.exp(s - m)
        denom = jnp.sum(p, axis=1, keepdims=True)
        o_ref[:, u * HD:(u + 1) * HD] = _dot(p, v) / denom


def _attn_call(q, k, v):
    qblk = 512
    grid = (NH // 2, SEQ // qblk)
    return pl.pallas_call(
        _attn_body,
        grid=grid,
        in_specs=[
            pl.BlockSpec((qblk, 2 * HD), lambda h2, qb: (qb, h2)),
            pl.BlockSpec((SEQ, 2 * HD), lambda h2, qb: (0, h2)),
            pl.BlockSpec((SEQ, 2 * HD), lambda h2, qb: (0, h2)),
        ],
        out_specs=pl.BlockSpec((qblk, 2 * HD), lambda h2, qb: (qb, h2)),
        out_shape=jax.ShapeDtypeStruct((SEQ, DIM), jnp.float32),
    )(q, k, v)


def _post_body(x_ref, o_ref, wo_ref, bo_ref, g_ref, b_ref, wg_ref, bg_ref,
               xa_ref, x2_ref, lg_ref):
    xa = x_ref[...] + _dot(o_ref[...], wo_ref[...], _DNT) + bo_ref[...]
    xa_ref[...] = xa
    mu = jnp.mean(xa, axis=1, keepdims=True)
    var = jnp.mean((xa - mu) * (xa - mu), axis=1, keepdims=True)
    x2 = (xa - mu) * lax.rsqrt(var + 1e-5) * g_ref[...] + b_ref[...]
    x2_ref[...] = x2
    lg_ref[...] = _dot(x2, wg_ref[...]) + bg_ref[...]


def _post_call(xf, o, wo, bo, g2, b2, wgp, bgp):
    blk = 256
    grid = (SEQ // blk,)
    spec_row = pl.BlockSpec((blk, DIM), lambda i: (i, 0))
    return pl.pallas_call(
        _post_body,
        grid=grid,
        in_specs=[
            spec_row,
            spec_row,
            pl.BlockSpec((DIM, DIM), lambda i: (0, 0)),
            pl.BlockSpec((1, DIM), lambda i: (0, 0)),
            pl.BlockSpec((1, DIM), lambda i: (0, 0)),
            pl.BlockSpec((1, DIM), lambda i: (0, 0)),
            pl.BlockSpec((DIM, 128), lambda i: (0, 0)),
            pl.BlockSpec((1, 128), lambda i: (0, 0)),
        ],
        out_specs=[spec_row, spec_row, pl.BlockSpec((blk, 128), lambda i: (i, 0))],
        out_shape=[
            jax.ShapeDtypeStruct((SEQ, DIM), jnp.float32),
            jax.ShapeDtypeStruct((SEQ, DIM), jnp.float32),
            jax.ShapeDtypeStruct((SEQ, 128), jnp.float32),
        ],
    )(xf, o, wo, bo, g2, b2, wgp, bgp)


def _route_body(lg_ref, dest_ref, gate_ref, gidx_ref, te_ref):
    f32 = jnp.float32
    lg = lg_ref[...]
    m = jnp.max(lg, axis=1, keepdims=True)
    ex = jnp.exp(lg - m)
    sm = jnp.sum(ex, axis=1, keepdims=True)
    gate_ref[...] = 1.0 / sm
    col = _fiota((SEQ, 128), 1)
    idxf = jnp.min(jnp.where(lg == m, col, 1e9), axis=1, keepdims=True)
    onehot = (col == idxf).astype(f32)

    nblk = SEQ // TILE
    rowb = jnp.floor(_fiota((nblk, SEQ), 1) * (1.0 / TILE))
    sel = (rowb == _fiota((nblk, SEQ), 0)).astype(f32)
    bc = _dot(sel, onehot)
    tri_b = (_fiota((nblk, nblk), 0)
             > _fiota((nblk, nblk), 1)).astype(f32)
    cum_bc = _dot(tri_b, bc)
    counts = jnp.sum(bc, axis=0, keepdims=True)
    padded = jnp.floor((counts + (TILE - 1.0)) * (1.0 / TILE)) * TILE
    lt = (_fiota((128, 128), 0)
          < _fiota((128, 128), 1)).astype(f32)
    offs = _dot(padded, lt)
    ends = offs + padded

    tri_t = (_fiota((TILE, TILE), 0)
             > _fiota((TILE, TILE), 1)).astype(f32)
    eye = (_fiota((TILE, TILE), 0)
           == _fiota((TILE, TILE), 1)).astype(f32)
    dest_chunks = []
    dest_rows = []
    for c in range(nblk):
        oh_c = onehot[c * TILE:(c + 1) * TILE]
        pos_c = _dot(tri_t, oh_c) + cum_bc[c:c + 1, :]
        d_c = jnp.sum(oh_c * (offs + pos_c), axis=1, keepdims=True)
        dest_chunks.append(d_c)
        dest_rows.append(jnp.sum(eye * d_c, axis=0, keepdims=True))
    dest = jnp.concatenate(dest_chunks, axis=0)
    dest_row = jnp.concatenate(dest_rows, axis=1)
    dest_ref[...] = dest.astype(jnp.int32)

    tok_row = _fiota((1, SEQ), 1)
    gidx_chunks = []
    for c in range(NTILES):
        slot = c * TILE + _fiota((TILE, 1), 0)
        eq = (dest_row == slot).astype(f32)
        gidx_chunks.append(jnp.sum(eq * tok_row, axis=1, keepdims=True))
    gidx_ref[...] = jnp.concatenate(gidx_chunks, axis=0).astype(jnp.int32)

    lane = _fiota((1, 128), 1)
    trow = _fiota((128, 1), 0) * float(TILE)
    cmp = jnp.where((ends <= trow) & (lane < float(NE)), 1.0, 0.0)
    te = jnp.minimum(jnp.sum(cmp, axis=1, keepdims=True), float(NE - 1))
    te_ref[...] = te.astype(jnp.int32)


def _route_call(logits):
    return pl.pallas_call(
        _route_body,
        in_specs=[pl.BlockSpec((SEQ, 128), lambda: (0, 0))],
        out_specs=[
            pl.BlockSpec((SEQ, 1), lambda: (0, 0)),
            pl.BlockSpec((SEQ, 1), lambda: (0, 0)),
            pl.BlockSpec((NPAD, 1), lambda: (0, 0)),
            pl.BlockSpec((128, 1), lambda: (0, 0)),
        ],
        out_shape=[
            jax.ShapeDtypeStruct((SEQ, 1), jnp.int32),
            jax.ShapeDtypeStruct((SEQ, 1), jnp.float32),
            jax.ShapeDtypeStruct((NPAD, 1), jnp.int32),
            jax.ShapeDtypeStruct((128, 1), jnp.int32),
        ],
    )(logits)


@functools.lru_cache(maxsize=None)
def _make_sc_gather(nrows, ncols, nidx, dtype):
    info = plsc.get_sparse_core_info()
    nw = info.num_cores * info.num_subcores
    per_w = nidx // nw
    mesh = plsc.VectorSubcoreMesh(core_axis_name="c", subcore_axis_name="s")

    @functools.partial(
        pl.kernel,
        mesh=mesh,
        out_type=jax.ShapeDtypeStruct((nidx, ncols), dtype),
        scratch_types=[
            pltpu.VMEM((per_w,), jnp.int32),
            pltpu.VMEM((per_w, ncols), dtype),
            pltpu.SemaphoreType.DMA,
        ],
    )
    def sc_gather(table_hbm, idx_hbm, out_hbm, idx_v, rows_v, sem):
        wid = lax.axis_index("s") * info.num_cores + lax.axis_index("c")
        base = wid * per_w
        pltpu.sync_copy(idx_hbm.at[pl.ds(base, per_w)], idx_v)
        pltpu.async_copy(table_hbm.at[idx_v], rows_v, sem).wait()
        pltpu.sync_copy(rows_v, out_hbm.at[pl.ds(base, per_w)])

    return sc_gather


def _sc_gather_rows(table, idx):
    return _make_sc_gather(table.shape[0], table.shape[1], idx.shape[0],
                           table.dtype)(table, idx)


def _moe_body(te_ref, gidx_ref, x2_ref, w1_ref, b1_ref, w2_ref, b2_ref, y_ref):
    t = pl.program_id(0)
    e = te_ref[t]
    oh = (lax.broadcasted_iota(jnp.int32, (1, NE), 1) == e).astype(jnp.float32)
    sel = (lax.broadcasted_iota(jnp.int32, (TILE, SEQ), 1)
           == gidx_ref[...]).astype(jnp.float32)
    xs = _dot(sel, x2_ref[...])
    h = _dot(xs, w1_ref[0]) + _dot(oh, b1_ref[...])
    h = 0.5 * h * (1.0 + lax.erf(h * 0.7071067811865476))
    y_ref[...] = _dot(h, w2_ref[0]) + _dot(oh, b2_ref[...])


def _moe_call(te, gidx, x2, w1, b1, w2, b2):
    grid_spec = pltpu.PrefetchScalarGridSpec(
        num_scalar_prefetch=1,
        grid=(NTILES,),
        in_specs=[
            pl.BlockSpec((TILE, 1), lambda t, te_r: (t, 0)),
            pl.BlockSpec((SEQ, DIM), lambda t, te_r: (0, 0)),
            pl.BlockSpec((1, DIM, DH), lambda t, te_r: (te_r[t], 0, 0)),
            pl.BlockSpec((NE, DH), lambda t, te_r: (0, 0)),
            pl.BlockSpec((1, DH, DIM), lambda t, te_r: (te_r[t], 0, 0)),
            pl.BlockSpec((NE, DIM), lambda t, te_r: (0, 0)),
        ],
        out_specs=pl.BlockSpec((TILE, DIM), lambda t, te_r: (t, 0)),
    )
    return pl.pallas_call(
        _moe_body,
        grid_spec=grid_spec,
        out_shape=jax.ShapeDtypeStruct((NPAD, DIM), jnp.float32),
    )(te, gidx, x2, w1, b1, w2, b2)


def _combine_body(xa_ref, gate_ref, yg_ref, o_ref):
    o_ref[...] = xa_ref[...] + gate_ref[...] * yg_ref[...]


def _combine_call(xa, gate, yg):
    blk = 256
    spec_row = pl.BlockSpec((blk, DIM), lambda i: (i, 0))
    return pl.pallas_call(
        _combine_body,
        grid=(SEQ // blk,),
        in_specs=[spec_row, pl.BlockSpec((blk, 1), lambda i: (i, 0)), spec_row],
        out_specs=spec_row,
        out_shape=jax.ShapeDtypeStruct((SEQ, DIM), jnp.float32),
    )(xa, gate, yg)


def kernel(x, gamma1, beta1, in_proj_w, in_proj_b, out_proj_w, out_proj_b,
           gamma2, beta2, Wg, bg, W1, b1, W2, b2):
    L, N, d = x.shape
    xf = x.reshape(L, d)

    q, k, v = _qkv_call(xf, gamma1.reshape(1, d), beta1.reshape(1, d),
                        in_proj_w, in_proj_b.reshape(1, 3 * d))
    o = _attn_call(q, k, v)

    wgp = jnp.concatenate([Wg, jnp.zeros((d, 128 - NE), jnp.float32)], axis=1)
    bgp = jnp.concatenate([bg, jnp.full((128 - NE,), -1e30, jnp.float32)])
    xa, x2, logits = _post_call(xf, o, out_proj_w, out_proj_b.reshape(1, d),
                                gamma2.reshape(1, d), beta2.reshape(1, d),
                                wgp, bgp.reshape(1, 128))

    dest, gate, gidx, te = _route_call(logits)

    ys = _moe_call(te.reshape(128)[:NTILES], gidx, x2, W1, b1, W2, b2)
    yg = _sc_gather_rows(ys, dest.reshape(L))

    out = _combine_call(xa, gate, yg)
    return out.reshape(L, N, d)


# speedup vs baseline: 1.9322x; 1.0210x over previous
import functools

import jax
import jax.numpy as jnp
from jax import lax
from jax.experimental import pallas as pl
from jax.experimental.pallas import tpu as pltpu
from jax.experimental.pallas import tpu_sc as plsc

DIM = 768
NH = 12
HD = 64
NE = 8
DH = 3072
SEQ = 2048
TILE = 128
NPAD = 3072
NTILES = NPAD // TILE
HSPLIT = 4
DHC = DH // HSPLIT

_DN = (((1,), (0,)), ((), ()))
_DNT = (((1,), (1,)), ((), ()))


def _dot(a, b, dn=_DN):
    return lax.dot_general(a, b, dn, preferred_element_type=jnp.float32)


def _fiota(shape, dim):
    return lax.broadcasted_iota(jnp.int32, shape, dim).astype(jnp.float32)


def _qkv_body(x_ref, g_ref, b_ref, w_ref, wb_ref, q_ref, k_ref, v_ref):
    x = x_ref[...]
    mu = jnp.mean(x, axis=1, keepdims=True)
    var = jnp.mean((x - mu) * (x - mu), axis=1, keepdims=True)
    xn = (x - mu) * lax.rsqrt(var + 1e-5) * g_ref[...] + b_ref[...]
    w = w_ref[...]
    wb = wb_ref[...]
    q_ref[...] = (_dot(xn, w[0:DIM], _DNT) + wb[0, 0:DIM]) * 0.125
    k_ref[...] = _dot(x, w[DIM:2 * DIM], _DNT) + wb[0, DIM:2 * DIM]
    v_ref[...] = _dot(x, w[2 * DIM:3 * DIM], _DNT) + wb[0, 2 * DIM:3 * DIM]


def _qkv_call(xf, g1, b1, w_in, b_in):
    blk = 256
    grid = (SEQ // blk,)
    spec_row = pl.BlockSpec((blk, DIM), lambda i: (i, 0))
    spec_full = pl.BlockSpec((DIM, DIM), lambda i: (0, 0))
    return pl.pallas_call(
        _qkv_body,
        grid=grid,
        in_specs=[
            spec_row,
            pl.BlockSpec((1, DIM), lambda i: (0, 0)),
            pl.BlockSpec((1, DIM), lambda i: (0, 0)),
            pl.BlockSpec((3 * DIM, DIM), lambda i: (0, 0)),
            pl.BlockSpec((1, 3 * DIM), lambda i: (0, 0)),
        ],
        out_specs=[spec_row, spec_row, spec_row],
        out_shape=[jax.ShapeDtypeStruct((SEQ, DIM), jnp.float32)] * 3,
    )(xf, g1, b1, w_in, b_in)


def _attn_body(q_ref, k_ref, v_ref, o_ref):
    for u in range(2):
        q = q_ref[:, u * HD:(u + 1) * HD]
        k = k_ref[:, u * HD:(u + 1) * HD]
        v = v_ref[:, u * HD:(u + 1) * HD]
        s = _dot(q, k, _DNT)
        m = jnp.max(s, axis=1, keepdims=True)
        p = jnp.exp(s - m)
        denom = jnp.sum(p, axis=1, keepdims=True)
        o_ref[:, u * HD:(u + 1) * HD] = _dot(p, v) / denom


def _attn_call(q, k, v):
    qblk = 512
    grid = (NH // 2, SEQ // qblk)
    return pl.pallas_call(
        _attn_body,
        grid=grid,
        in_specs=[
            pl.BlockSpec((qblk, 2 * HD), lambda h2, qb: (qb, h2)),
            pl.BlockSpec((SEQ, 2 * HD), lambda h2, qb: (0, h2)),
            pl.BlockSpec((SEQ, 2 * HD), lambda h2, qb: (0, h2)),
        ],
        out_specs=pl.BlockSpec((qblk, 2 * HD), lambda h2, qb: (qb, h2)),
        out_shape=jax.ShapeDtypeStruct((SEQ, DIM), jnp.float32),
    )(q, k, v)


def _post_body(x_ref, o_ref, wo_ref, bo_ref, g_ref, b_ref, wg_ref, bg_ref,
               xa_ref, x2_ref, lg_ref):
    xa = x_ref[...] + _dot(o_ref[...], wo_ref[...], _DNT) + bo_ref[...]
    xa_ref[...] = xa
    mu = jnp.mean(xa, axis=1, keepdims=True)
    var = jnp.mean((xa - mu) * (xa - mu), axis=1, keepdims=True)
    x2 = (xa - mu) * lax.rsqrt(var + 1e-5) * g_ref[...] + b_ref[...]
    x2_ref[...] = x2
    lg_ref[...] = _dot(x2, wg_ref[...]) + bg_ref[...]


def _post_call(xf, o, wo, bo, g2, b2, wgp, bgp):
    blk = 256
    grid = (SEQ // blk,)
    spec_row = pl.BlockSpec((blk, DIM), lambda i: (i, 0))
    return pl.pallas_call(
        _post_body,
        grid=grid,
        in_specs=[
            spec_row,
            spec_row,
            pl.BlockSpec((DIM, DIM), lambda i: (0, 0)),
            pl.BlockSpec((1, DIM), lambda i: (0, 0)),
            pl.BlockSpec((1, DIM), lambda i: (0, 0)),
            pl.BlockSpec((1, DIM), lambda i: (0, 0)),
            pl.BlockSpec((DIM, 128), lambda i: (0, 0)),
            pl.BlockSpec((1, 128), lambda i: (0, 0)),
        ],
        out_specs=[spec_row, spec_row, pl.BlockSpec((blk, 128), lambda i: (i, 0))],
        out_shape=[
            jax.ShapeDtypeStruct((SEQ, DIM), jnp.float32),
            jax.ShapeDtypeStruct((SEQ, DIM), jnp.float32),
            jax.ShapeDtypeStruct((SEQ, 128), jnp.float32),
        ],
    )(xf, o, wo, bo, g2, b2, wgp, bgp)


def _route_body(lg_ref, destr_ref, gate_ref, te_ref):
    f32 = jnp.float32
    lg = lg_ref[...]
    m = jnp.max(lg, axis=1, keepdims=True)
    ex = jnp.exp(lg - m)
    sm = jnp.sum(ex, axis=1, keepdims=True)
    gate_ref[...] = 1.0 / sm
    col = _fiota((SEQ, 128), 1)
    idxf = jnp.min(jnp.where(lg == m, col, 1e9), axis=1, keepdims=True)
    onehot = (col == idxf).astype(f32)

    nblk = SEQ // TILE
    rowb = jnp.floor(_fiota((nblk, SEQ), 1) * (1.0 / TILE))
    sel = (rowb == _fiota((nblk, SEQ), 0)).astype(f32)
    bc = _dot(sel, onehot)
    tri_b = (_fiota((nblk, nblk), 0)
             > _fiota((nblk, nblk), 1)).astype(f32)
    cum_bc = _dot(tri_b, bc)
    counts = jnp.sum(bc, axis=0, keepdims=True)
    padded = jnp.floor((counts + (TILE - 1.0)) * (1.0 / TILE)) * TILE
    lt = (_fiota((128, 128), 0)
          < _fiota((128, 128), 1)).astype(f32)
    offs = _dot(padded, lt)
    ends = offs + padded

    tri_t = (_fiota((TILE, TILE), 0)
             > _fiota((TILE, TILE), 1)).astype(f32)
    eye = (_fiota((TILE, TILE), 0)
           == _fiota((TILE, TILE), 1)).astype(f32)
    dest_rows = []
    for c in range(nblk):
        oh_c = onehot[c * TILE:(c + 1) * TILE]
        pos_c = _dot(tri_t, oh_c) + cum_bc[c:c + 1, :]
        d_c = jnp.sum(oh_c * (offs + pos_c), axis=1, keepdims=True)
        dest_rows.append(jnp.sum(eye * d_c, axis=0, keepdims=True))
    dest_row = jnp.concatenate(dest_rows, axis=1)
    destr_ref[...] = dest_row.astype(jnp.int32)

    lane = _fiota((1, 128), 1)
    trow = _fiota((128, 1), 0) * float(TILE)
    cmp = jnp.where((ends <= trow) & (lane < float(NE)), 1.0, 0.0)
    te = jnp.minimum(jnp.sum(cmp, axis=1, keepdims=True), float(NE - 1))
    te_ref[...] = te.astype(jnp.int32)


def _route_call(logits):
    return pl.pallas_call(
        _route_body,
        in_specs=[pl.BlockSpec((SEQ, 128), lambda: (0, 0))],
        out_specs=[
            pl.BlockSpec((1, SEQ), lambda: (0, 0)),
            pl.BlockSpec((SEQ, 1), lambda: (0, 0)),
            pl.BlockSpec((128, 1), lambda: (0, 0)),
        ],
        out_shape=[
            jax.ShapeDtypeStruct((1, SEQ), jnp.int32),
            jax.ShapeDtypeStruct((SEQ, 1), jnp.float32),
            jax.ShapeDtypeStruct((128, 1), jnp.int32),
        ],
    )(logits)


@functools.lru_cache(maxsize=None)
def _make_sc_gather(nrows, ncols, nidx, dtype):
    info = plsc.get_sparse_core_info()
    nw = info.num_cores * info.num_subcores
    per_w = nidx // nw
    mesh = plsc.VectorSubcoreMesh(core_axis_name="c", subcore_axis_name="s")

    @functools.partial(
        pl.kernel,
        mesh=mesh,
        out_type=jax.ShapeDtypeStruct((nidx, ncols), dtype),
        scratch_types=[
            pltpu.VMEM((per_w,), jnp.int32),
            pltpu.VMEM((per_w, ncols), dtype),
            pltpu.SemaphoreType.DMA,
        ],
    )
    def sc_gather(table_hbm, idx_hbm, out_hbm, idx_v, rows_v, sem):
        wid = lax.axis_index("s") * info.num_cores + lax.axis_index("c")
        base = wid * per_w
        pltpu.sync_copy(idx_hbm.at[pl.ds(base, per_w)], idx_v)
        pltpu.async_copy(table_hbm.at[idx_v], rows_v, sem).wait()
        pltpu.sync_copy(rows_v, out_hbm.at[pl.ds(base, per_w)])

    return sc_gather


def _sc_gather_rows(table, idx):
    return _make_sc_gather(table.shape[0], table.shape[1], idx.shape[0],
                           table.dtype)(table, idx)


def _moe_body(te_ref, destr_ref, x2_ref, w1_ref, b1_ref, w2_ref, b2_ref, y_ref):
    t = pl.program_id(0)
    e = te_ref[t]
    oh = (lax.broadcasted_iota(jnp.int32, (1, NE), 1) == e).astype(jnp.float32)
    sel = (lax.broadcasted_iota(jnp.int32, (TILE, SEQ), 0) + t * TILE
           == destr_ref[...]).astype(jnp.float32)
    xs = _dot(sel, x2_ref[...])
    h = _dot(xs, w1_ref[0]) + _dot(oh, b1_ref[...])
    h = 0.5 * h * (1.0 + lax.erf(h * 0.7071067811865476))
    y_ref[...] = _dot(h, w2_ref[0]) + _dot(oh, b2_ref[...])


def _moe_call(te, destr, x2, w1, b1, w2, b2):
    grid_spec = pltpu.PrefetchScalarGridSpec(
        num_scalar_prefetch=1,
        grid=(NTILES,),
        in_specs=[
            pl.BlockSpec((1, SEQ), lambda t, te_r: (0, 0)),
            pl.BlockSpec((SEQ, DIM), lambda t, te_r: (0, 0)),
            pl.BlockSpec((1, DIM, DH), lambda t, te_r: (te_r[t], 0, 0)),
            pl.BlockSpec((NE, DH), lambda t, te_r: (0, 0)),
            pl.BlockSpec((1, DH, DIM), lambda t, te_r: (te_r[t], 0, 0)),
            pl.BlockSpec((NE, DIM), lambda t, te_r: (0, 0)),
        ],
        out_specs=pl.BlockSpec((TILE, DIM), lambda t, te_r: (t, 0)),
    )
    return pl.pallas_call(
        _moe_body,
        grid_spec=grid_spec,
        out_shape=jax.ShapeDtypeStruct((NPAD, DIM), jnp.float32),
    )(te, destr, x2, w1, b1, w2, b2)


def _combine_body(xa_ref, gate_ref, yg_ref, o_ref):
    o_ref[...] = xa_ref[...] + gate_ref[...] * yg_ref[...]


def _combine_call(xa, gate, yg):
    blk = 256
    spec_row = pl.BlockSpec((blk, DIM), lambda i: (i, 0))
    return pl.pallas_call(
        _combine_body,
        grid=(SEQ // blk,),
        in_specs=[spec_row, pl.BlockSpec((blk, 1), lambda i: (i, 0)), spec_row],
        out_specs=spec_row,
        out_shape=jax.ShapeDtypeStruct((SEQ, DIM), jnp.float32),
    )(xa, gate, yg)


def kernel(x, gamma1, beta1, in_proj_w, in_proj_b, out_proj_w, out_proj_b,
           gamma2, beta2, Wg, bg, W1, b1, W2, b2):
    L, N, d = x.shape
    xf = x.reshape(L, d)

    q, k, v = _qkv_call(xf, gamma1.reshape(1, d), beta1.reshape(1, d),
                        in_proj_w, in_proj_b.reshape(1, 3 * d))
    o = _attn_call(q, k, v)

    wgp = jnp.concatenate([Wg, jnp.zeros((d, 128 - NE), jnp.float32)], axis=1)
    bgp = jnp.concatenate([bg, jnp.full((128 - NE,), -1e30, jnp.float32)])
    xa, x2, logits = _post_call(xf, o, out_proj_w, out_proj_b.reshape(1, d),
                                gamma2.reshape(1, d), beta2.reshape(1, d),
                                wgp, bgp.reshape(1, 128))

    dest_row, gate, te = _route_call(logits)

    ys = _moe_call(te.reshape(128)[:NTILES], dest_row, x2, W1, b1, W2, b2)
    yg = _sc_gather_rows(ys, dest_row.reshape(L))

    out = _combine_call(xa, gate, yg)
    return out.reshape(L, N, d)
